# baseline (speedup 1.0000x reference)
import functools

import jax
import jax.numpy as jnp
from jax import lax
from jax.experimental import pallas as pl
from jax.experimental.pallas import tpu as pltpu
from jax.experimental.pallas import tpu_sc as plsc

P = 32
ALPHA = 16
VOCAB = 1000000
VOCAB_PAD = 1000064
B = 1024
L = 50
N_IDS = B * L
W_WORDS = P * VOCAB_PAD

NB = 8
CHUNK = 128
MAXROWS = 13 * P


_DT_CH = 61 * 128
_DT_COLS = 16 * _DT_CH


def _make_detile():
    mesh = plsc.VectorSubcoreMesh(core_axis_name="c", subcore_axis_name="s")

    @functools.partial(
        pl.kernel,
        mesh=mesh,
        out_type=jax.ShapeDtypeStruct((W_WORDS,), jnp.float32),
        compiler_params=pltpu.CompilerParams(needs_layout_passes=False),
        scratch_types=[
            pltpu.VMEM((2, 8, _DT_CH), jnp.float32),
            pltpu.SemaphoreType.DMA,
            pltpu.SemaphoreType.DMA,
        ],
    )
    def detile(w_hbm, wlin_hbm, tmp_v, sem_in, sem_out):
        wid = lax.axis_index("s") * 2 + lax.axis_index("c")
        band = wid % 4
        g = wid // 4
        col0 = g * _DT_COLS

        def in_refs(k, buf, size):
            return (
                w_hbm.at[
                    pl.ds(band * 8, 8), pl.ds(col0 + k * _DT_CH, size)
                ],
                tmp_v.at[buf, :, pl.ds(0, size)],
            )

        def out_refs(k, buf, i, size):
            return (
                tmp_v.at[buf, i, pl.ds(0, size)],
                wlin_hbm.at[
                    pl.ds(
                        (band * 8 + i) * VOCAB_PAD + col0 + k * _DT_CH, size
                    )
                ],
            )

        def run(sizes):
            n = len(sizes)
            pltpu.async_copy(*in_refs(0, 0, sizes[0]), sem_in)
            for k in range(n):
                buf = k % 2
                pltpu.make_async_copy(*in_refs(k, buf, sizes[k]), sem_in).wait()
                for i in range(8):
                    pltpu.async_copy(*out_refs(k, buf, i, sizes[k]), sem_out)
                if k >= 1:
                    for i in range(8):
                        pltpu.make_async_copy(
                            *out_refs(k - 1, 1 - buf, i, sizes[k - 1]), sem_out
                        ).wait()
                if k + 1 < n:
                    pltpu.async_copy(*in_refs(k + 1, 1 - buf, sizes[k + 1]), sem_in)
            for i in range(8):
                pltpu.make_async_copy(
                    *out_refs(n - 1, (n - 1) % 2, i, sizes[n - 1]), sem_out
                ).wait()

        @pl.when(g < 7)
        def _():
            run((_DT_CH,) * 16)

        @pl.when(g == 7)
        def _():
            run((_DT_CH,) * 16 + (512,))

    return detile


_detile_call = _make_detile()


def _make_gather():
    mesh = plsc.VectorSubcoreMesh(core_axis_name="c", subcore_axis_name="s")

    @functools.partial(
        pl.kernel,
        mesh=mesh,
        out_type=jax.ShapeDtypeStruct((L * P, B), jnp.float32),
        compiler_params=pltpu.CompilerParams(needs_layout_passes=False),
        scratch_types=[
            pltpu.VMEM((13 * 128,), jnp.int32),
            pltpu.VMEM((MAXROWS * 128,), jnp.int32),
            pltpu.VMEM((MAXROWS, 128), jnp.float32),
            pltpu.SemaphoreType.DMA,
        ],
    )
    def gather(ids_hbm, w_hbm, u_hbm, ids_v, idx_v, u_v, sem):
        wid = lax.axis_index("s") * 2 + lax.axis_index("c")
        wb = wid % NB
        wl = wid // NB

        def phase(l0, nl):
            pltpu.sync_copy(
                ids_hbm.at[pl.ds(wb * (L * 128) + l0 * 128, nl * 128)],
                ids_v.at[pl.ds(0, nl * 128)],
            )

            def build_fire(j, carry):
                l_rel = j // P
                p = j % P
                off = p * VOCAB_PAD
                for h in range(8):
                    ids16 = ids_v[pl.ds(l_rel * 128 + h * 16, 16)]
                    idx_v[pl.ds(j * 128 + h * 16, 16)] = ids16 + off
                s = pl.ds(pl.multiple_of(j * CHUNK, CHUNK), CHUNK)
                pltpu.async_copy(w_hbm.at[idx_v.at[s]], u_v.at[j], sem)
                return carry

            def drain(j, carry):
                s = pl.ds(pl.multiple_of(j * CHUNK, CHUNK), CHUNK)
                pltpu.make_async_copy(
                    w_hbm.at[idx_v.at[s]], u_v.at[j], sem
                ).wait()
                return carry

            lax.fori_loop(0, nl * P, build_fire, 0, unroll=False)
            lax.fori_loop(0, nl * P, drain, 0, unroll=False)
            pltpu.sync_copy(
                u_v.at[pl.ds(0, nl * P), :],
                u_hbm.at[pl.ds(l0 * P, nl * P), pl.ds(wb * 128, 128)],
            )

        @pl.when(wl < 2)
        def _():
            phase(wl * 13, 13)

        @pl.when(wl >= 2)
        def _():
            phase(26 + (wl - 2) * 12, 12)

    return gather


_gather_call = _make_gather()

_RB = 64
_NI = L * P // _RB


def _dense_body(u_ref, sp_ref, m_ref, fh_ref, y_ref):
    u = u_ref[...]
    m = (u >= sp_ref[...]).astype(jnp.float32)
    fh_ref[...] = m
    y_ref[...] = lax.dot_general(
        m_ref[...], m * u, (((1,), (0,)), ((), ())),
        preferred_element_type=jnp.float32,
    )


def kernel(input_ids, W_emb, sp, f_param):
    ids_lin = (
        input_ids.astype(jnp.int32)
        .T.reshape(L, NB, 128)
        .transpose(1, 0, 2)
        .reshape(-1)
    )
    w_flat = _detile_call(W_emb)
    tail = W_emb[:, 128 * (VOCAB // 128):]
    pos = (
        jnp.arange(P, dtype=jnp.int32)[:, None] * VOCAB_PAD
        + 128 * (VOCAB // 128)
        + jnp.arange(VOCAB - 128 * (VOCAB // 128), dtype=jnp.int32)[None, :]
    )
    w_flat = w_flat.at[pos.reshape(-1)].set(tail.reshape(-1))
    u2 = _gather_call(ids_lin, w_flat)

    sp_rep = jnp.tile(sp.astype(jnp.float32), 2).reshape(_RB, 1)
    eye = jnp.eye(P, dtype=jnp.float32)
    E = (eye[:, :, None] * f_param.astype(jnp.float32)[None, :, :]).reshape(
        P, P * ALPHA
    )
    M = jnp.kron(jnp.eye(2, dtype=jnp.float32), E.T)

    fh2, yt = pl.pallas_call(
        _dense_body,
        grid=(_NI,),
        in_specs=[
            pl.BlockSpec((_RB, B), lambda i: (i, 0)),
            pl.BlockSpec((_RB, 1), lambda i: (0, 0)),
            pl.BlockSpec((_RB * ALPHA, _RB), lambda i: (0, 0)),
        ],
        out_specs=[
            pl.BlockSpec((_RB, B), lambda i: (i, 0)),
            pl.BlockSpec((_RB * ALPHA, B), lambda i: (i, 0)),
        ],
        out_shape=[
            jax.ShapeDtypeStruct((L * P, B), jnp.float32),
            jax.ShapeDtypeStruct((L * P * ALPHA, B), jnp.float32),
        ],
    )(u2, sp_rep, M)

    u = u2.reshape(L, P, B).transpose(2, 0, 1)
    f_hard = fh2.reshape(L, P, B).transpose(2, 0, 1)
    y = yt.reshape(L, P, ALPHA, B).transpose(3, 0, 1, 2)
    return (u, f_hard, y)


# speedup vs baseline: 2.1739x; 2.1739x over previous
import functools

import jax
import jax.numpy as jnp
from jax import lax
from jax.experimental import pallas as pl
from jax.experimental.pallas import tpu as pltpu
from jax.experimental.pallas import tpu_sc as plsc

P = 32
ALPHA = 16
VOCAB = 1000000
VOCAB_PAD = 1000064
B = 1024
L = 50
N_IDS = B * L
W_WORDS = P * VOCAB_PAD

NB = 8
CHUNK = 128
MAXROWS = 13 * P


_DT_CH = 61 * 128
_DT_COLS = 16 * _DT_CH


def _make_detile():
    mesh = plsc.VectorSubcoreMesh(core_axis_name="c", subcore_axis_name="s")

    @functools.partial(
        pl.kernel,
        mesh=mesh,
        out_type=jax.ShapeDtypeStruct((W_WORDS,), jnp.float32),
        compiler_params=pltpu.CompilerParams(needs_layout_passes=False),
        scratch_types=[
            pltpu.VMEM((2, 8, _DT_CH), jnp.float32),
            pltpu.SemaphoreType.DMA,
            pltpu.SemaphoreType.DMA,
        ],
    )
    def detile(w_hbm, wlin_hbm, tmp_v, sem_in, sem_out):
        wid = lax.axis_index("s") * 2 + lax.axis_index("c")
        band = wid % 4
        g = wid // 4
        col0 = g * _DT_COLS

        def in_refs(k, buf, size):
            return (
                w_hbm.at[
                    pl.ds(band * 8, 8), pl.ds(col0 + k * _DT_CH, size)
                ],
                tmp_v.at[buf, :, pl.ds(0, size)],
            )

        def out_refs(k, buf, i, size):
            return (
                tmp_v.at[buf, i, pl.ds(0, size)],
                wlin_hbm.at[
                    pl.ds(
                        (band * 8 + i) * VOCAB_PAD + col0 + k * _DT_CH, size
                    )
                ],
            )

        def run(sizes):
            n = len(sizes)
            pltpu.async_copy(*in_refs(0, 0, sizes[0]), sem_in)
            for k in range(n):
                buf = k % 2
                pltpu.make_async_copy(*in_refs(k, buf, sizes[k]), sem_in).wait()
                for i in range(8):
                    pltpu.async_copy(*out_refs(k, buf, i, sizes[k]), sem_out)
                if k >= 1:
                    for i in range(8):
                        pltpu.make_async_copy(
                            *out_refs(k - 1, 1 - buf, i, sizes[k - 1]), sem_out
                        ).wait()
                if k + 1 < n:
                    pltpu.async_copy(*in_refs(k + 1, 1 - buf, sizes[k + 1]), sem_in)
            for i in range(8):
                pltpu.make_async_copy(
                    *out_refs(n - 1, (n - 1) % 2, i, sizes[n - 1]), sem_out
                ).wait()

        @pl.when(g < 7)
        def _():
            run((_DT_CH,) * 16)

        @pl.when(g == 7)
        def _():
            run((_DT_CH,) * 16 + (512,))

    return detile


_detile_call = _make_detile()


def _make_gather():
    mesh = plsc.VectorSubcoreMesh(core_axis_name="c", subcore_axis_name="s")

    @functools.partial(
        pl.kernel,
        mesh=mesh,
        out_type=jax.ShapeDtypeStruct((L * P, B), jnp.float32),
        compiler_params=pltpu.CompilerParams(needs_layout_passes=False),
        scratch_types=[
            pltpu.VMEM((13 * 128,), jnp.int32),
            pltpu.VMEM((MAXROWS * 128,), jnp.int32),
            pltpu.VMEM((MAXROWS, 128), jnp.float32),
            pltpu.VMEM((P * 128,), jnp.float32),
            pltpu.SemaphoreType.DMA,
        ],
    )
    def gather(ids_hbm, w_hbm, tail_hbm, u_hbm, ids_v, idx_v, u_v, tail_v, sem):
        wid = lax.axis_index("s") * 2 + lax.axis_index("c")
        wb = wid % NB
        wl = wid // NB

        def phase(l0, nl):
            pltpu.sync_copy(
                ids_hbm.at[pl.ds(wb * (L * 128) + l0 * 128, nl * 128)],
                ids_v.at[pl.ds(0, nl * 128)],
            )
            pltpu.sync_copy(tail_hbm, tail_v)
            tail_base = 128 * (VOCAB // 128)
            lanes16 = lax.iota(jnp.int32, 16)

            def build_fire(j, carry):
                l_rel = j // P
                p = j % P
                off = p * VOCAB_PAD
                for h in range(8):
                    ids16 = ids_v[pl.ds(l_rel * 128 + h * 16, 16)]
                    safe = jnp.where(ids16 >= tail_base, 0, ids16)
                    idx_v[pl.ds(j * 128 + h * 16, 16)] = safe + off
                s = pl.ds(pl.multiple_of(j * CHUNK, CHUNK), CHUNK)
                pltpu.async_copy(w_hbm.at[idx_v.at[s]], u_v.at[j], sem)
                return carry

            def drain(j, carry):
                s = pl.ds(pl.multiple_of(j * CHUNK, CHUNK), CHUNK)
                pltpu.make_async_copy(
                    w_hbm.at[idx_v.at[s]], u_v.at[j], sem
                ).wait()
                return carry

            def fixup(gi, carry):
                ids16 = ids_v[pl.ds(gi * 16, 16)]
                mask = ids16 >= tail_base

                @pl.when(jnp.any(mask))
                def _():
                    l_rel = gi // 8
                    h = gi % 8
                    toff = ids16 - tail_base
                    cols = h * 16 + lanes16
                    for p in range(P):
                        tv = plsc.load_gather(
                            tail_v, [toff + p * 128], mask=mask
                        )
                        rows = jnp.broadcast_to(l_rel * P + p, (16,))
                        plsc.store_scatter(
                            u_v, [rows, cols], tv, mask=mask
                        )

                return carry

            lax.fori_loop(0, nl * P, build_fire, 0, unroll=False)
            lax.fori_loop(0, nl * P, drain, 0, unroll=False)
            lax.fori_loop(0, nl * 8, fixup, 0, unroll=False)
            pltpu.sync_copy(
                u_v.at[pl.ds(0, nl * P), :],
                u_hbm.at[pl.ds(l0 * P, nl * P), pl.ds(wb * 128, 128)],
            )

        @pl.when(wl < 2)
        def _():
            phase(wl * 13, 13)

        @pl.when(wl >= 2)
        def _():
            phase(26 + (wl - 2) * 12, 12)

    return gather


_gather_call = _make_gather()

_RB = 64
_NI = L * P // _RB


def _dense_body(u_ref, sp_ref, m_ref, fh_ref, y_ref):
    u = u_ref[...]
    m = (u >= sp_ref[...]).astype(jnp.float32)
    fh_ref[...] = m
    y_ref[...] = lax.dot_general(
        m_ref[...], m * u, (((1,), (0,)), ((), ())),
        preferred_element_type=jnp.float32,
    )


def kernel(input_ids, W_emb, sp, f_param):
    ids_lin = (
        input_ids.astype(jnp.int32)
        .T.reshape(L, NB, 128)
        .transpose(1, 0, 2)
        .reshape(-1)
    )
    w_flat = _detile_call(W_emb)
    tail_lin = lax.pad(
        W_emb[:, 128 * (VOCAB // 128):],
        jnp.float32(0),
        ((0, 0, 0), (0, 64, 0)),
    ).reshape(P * 128)
    u2 = _gather_call(ids_lin, w_flat, tail_lin)

    sp_rep = jnp.tile(sp.astype(jnp.float32), 2).reshape(_RB, 1)
    eye = jnp.eye(P, dtype=jnp.float32)
    E = (eye[:, :, None] * f_param.astype(jnp.float32)[None, :, :]).reshape(
        P, P * ALPHA
    )
    M = jnp.kron(jnp.eye(2, dtype=jnp.float32), E.T)

    fh2, yt = pl.pallas_call(
        _dense_body,
        grid=(_NI,),
        in_specs=[
            pl.BlockSpec((_RB, B), lambda i: (i, 0)),
            pl.BlockSpec((_RB, 1), lambda i: (0, 0)),
            pl.BlockSpec((_RB * ALPHA, _RB), lambda i: (0, 0)),
        ],
        out_specs=[
            pl.BlockSpec((_RB, B), lambda i: (i, 0)),
            pl.BlockSpec((_RB * ALPHA, B), lambda i: (i, 0)),
        ],
        out_shape=[
            jax.ShapeDtypeStruct((L * P, B), jnp.float32),
            jax.ShapeDtypeStruct((L * P * ALPHA, B), jnp.float32),
        ],
    )(u2, sp_rep, M)

    u = u2.reshape(L, P, B).transpose(2, 0, 1)
    f_hard = fh2.reshape(L, P, B).transpose(2, 0, 1)
    y = yt.reshape(L, P, ALPHA, B).transpose(3, 0, 1, 2)
    return (u, f_hard, y)
